# Initial kernel scaffold; baseline (speedup 1.0000x reference)
import jax
import jax.numpy as jnp
from jax.experimental import pallas as pl


def kernel(x, shifts):
    raise NotImplementedError("write your pallas kernel here")


# speedup vs baseline: 1.8437x; 1.8437x over previous
import jax
import jax.numpy as jnp
from jax import lax
from jax.experimental import pallas as pl
from jax.experimental.pallas import tpu as pltpu
from jax.experimental.pallas import tpu_sc as plsc

_B, _T, _D = 32, 512, 1024
_SENT = 99
_C = 32
_NCHUNK = _T // _C
_NC, _NS = 2, 16


def _sc_body(x_hbm, sh_hbm, out_hbm, shp_v, win_v, mask_v, buf_v, sem_in):
    b = lax.axis_index("s") * _NC + lax.axis_index("c")
    iota = lax.iota(jnp.int32, 16)

    shp_v[pl.ds(0, 16)] = iota * 0 + _SENT
    shp_v[pl.ds(_T + 16, 16)] = iota * 0 + _SENT
    pltpu.sync_copy(sh_hbm.at[pl.ds(b * _T, _T)], shp_v.at[pl.ds(16, _T)])

    def win_chunk(j, carry):
        nt = iota + j * 16
        base = nt + 16
        t2 = plsc.load_gather(shp_v, [base + 2])
        t1 = plsc.load_gather(shp_v, [base + 1])
        t0 = plsc.load_gather(shp_v, [base])
        tm1 = plsc.load_gather(shp_v, [base - 1])
        tm2 = plsc.load_gather(shp_v, [base - 2])
        c2 = t2 == -2
        c1 = t1 == -1
        c0 = t0 == 0
        cm1 = tm1 == 1
        cm2 = tm2 == 2
        off = jnp.where(
            c2, 2,
            jnp.where(c1, 1,
                      jnp.where(c0, 0,
                                jnp.where(cm1, -1, jnp.where(cm2, -2, 0)))))
        ne = c2 | c1 | c0 | cm1 | cm2
        win_v[pl.ds(j * 16, 16)] = nt + off + b * _T
        mask_v[pl.ds(j * 16, 16)] = jnp.where(ne, 1.0, 0.0).astype(jnp.float32)
        return carry

    lax.fori_loop(0, _T // 16, win_chunk, 0)

    def do_chunk(c, carry):
        idx = win_v.at[pl.ds(c * _C, _C)]
        pltpu.async_copy(x_hbm.at[idx], buf_v, sem_in).wait()

        def mul_row(r, rc):
            m = plsc.load_gather(mask_v, [iota * 0 + (c * _C + r)])
            for v in range(_D // 16):
                buf_v[r, pl.ds(v * 16, 16)] = buf_v[r, pl.ds(v * 16, 16)] * m
            return rc

        lax.fori_loop(0, _C, mul_row, 0)
        pltpu.sync_copy(buf_v, out_hbm.at[pl.ds(b * _T + c * _C, _C)])
        return carry

    lax.fori_loop(0, _NCHUNK, do_chunk, 0)


@jax.jit
def kernel(x, shifts):
    xf = x.reshape(_B * _T, _D)
    shf = shifts.reshape(_B * _T)
    mesh = plsc.VectorSubcoreMesh(core_axis_name="c", subcore_axis_name="s")
    out = pl.kernel(
        _sc_body,
        out_type=jax.ShapeDtypeStruct((_B * _T, _D), jnp.float32),
        mesh=mesh,
        compiler_params=pltpu.CompilerParams(needs_layout_passes=False),
        scratch_types=[
            pltpu.VMEM((_T + 32,), jnp.int32),
            pltpu.VMEM((_T,), jnp.int32),
            pltpu.VMEM((_T,), jnp.float32),
            pltpu.VMEM((_C, _D), jnp.float32),
            pltpu.SemaphoreType.DMA,
        ],
    )(xf, shf)
    return out.reshape(_B, _T, _D)


# speedup vs baseline: 2.0294x; 1.1007x over previous
import jax
import jax.numpy as jnp
from jax import lax
from jax.experimental import pallas as pl
from jax.experimental.pallas import tpu as pltpu
from jax.experimental.pallas import tpu_sc as plsc

_B, _T, _D = 32, 512, 1024
_SENT = 99
_C = 32
_NBUF = 2
_NCHUNK = _T // _C
_NC, _NS = 2, 16


def _sc_body(x_hbm, sh_hbm, out_hbm, shp_v, win_v, mask_v, buf_v,
             gsem0, gsem1, ssem0, ssem1):
    b = lax.axis_index("s") * _NC + lax.axis_index("c")
    iota = lax.iota(jnp.int32, 16)

    shp_v[pl.ds(0, 16)] = iota * 0 + _SENT
    shp_v[pl.ds(_T + 16, 16)] = iota * 0 + _SENT
    pltpu.sync_copy(sh_hbm.at[pl.ds(b * _T, _T)], shp_v.at[pl.ds(16, _T)])

    def win_chunk(j, carry):
        nt = iota + j * 16
        base = nt + 16
        t2 = plsc.load_gather(shp_v, [base + 2])
        t1 = plsc.load_gather(shp_v, [base + 1])
        t0 = plsc.load_gather(shp_v, [base])
        tm1 = plsc.load_gather(shp_v, [base - 1])
        tm2 = plsc.load_gather(shp_v, [base - 2])
        c2 = t2 == -2
        c1 = t1 == -1
        c0 = t0 == 0
        cm1 = tm1 == 1
        cm2 = tm2 == 2
        off = jnp.where(
            c2, 2,
            jnp.where(c1, 1,
                      jnp.where(c0, 0,
                                jnp.where(cm1, -1, jnp.where(cm2, -2, 0)))))
        ne = c2 | c1 | c0 | cm1 | cm2
        win_v[pl.ds(j * 16, 16)] = nt + off + b * _T
        mask_v[pl.ds(j * 16, 16)] = jnp.where(ne, 1.0, 0.0).astype(jnp.float32)
        return carry

    lax.fori_loop(0, _T // 16, win_chunk, 0)

    gsems = (gsem0, gsem1)
    ssems = (ssem0, ssem1)

    def g_start(c, s):
        pltpu.make_async_copy(
            x_hbm.at[win_v.at[pl.ds(c * _C, _C)]], buf_v.at[s], gsems[s]
        ).start()

    def g_wait(c, s):
        pltpu.make_async_copy(
            x_hbm.at[win_v.at[pl.ds(c * _C, _C)]], buf_v.at[s], gsems[s]
        ).wait()

    def s_start(c, s):
        pltpu.make_async_copy(
            buf_v.at[s], out_hbm.at[pl.ds(b * _T + c * _C, _C)], ssems[s]
        ).start()

    def s_wait(c, s):
        pltpu.make_async_copy(
            buf_v.at[s], out_hbm.at[pl.ds(b * _T + c * _C, _C)], ssems[s]
        ).wait()

    g_start(0, 0)

    def superstep(i, carry):
        for s in range(_NBUF):
            c = i * _NBUF + s
            g_wait(c, s)

            def mul_row(r, rc):
                m = plsc.load_gather(mask_v, [iota * 0 + (c * _C + r)])
                for v in range(_D // 16):
                    buf_v[s, r, pl.ds(v * 16, 16)] = (
                        buf_v[s, r, pl.ds(v * 16, 16)] * m)
                return rc

            lax.fori_loop(0, _C, mul_row, 0)
            s_start(c, s)
            ns = (s + 1) % _NBUF

            @pl.when(c + 1 < _NCHUNK)
            def _():
                @pl.when(c + 1 - _NBUF >= 0)
                def _():
                    s_wait(c + 1 - _NBUF, ns)

                g_start(c + 1, ns)
        return carry

    lax.fori_loop(0, _NCHUNK // _NBUF, superstep, 0)
    for s in range(_NBUF):
        s_wait(_NCHUNK - _NBUF + s, s)


@jax.jit
def kernel(x, shifts):
    xf = x.reshape(_B * _T, _D)
    shf = shifts.reshape(_B * _T)
    mesh = plsc.VectorSubcoreMesh(core_axis_name="c", subcore_axis_name="s")
    out = pl.kernel(
        _sc_body,
        out_type=jax.ShapeDtypeStruct((_B * _T, _D), jnp.float32),
        mesh=mesh,
        compiler_params=pltpu.CompilerParams(needs_layout_passes=False),
        scratch_types=[
            pltpu.VMEM((_T + 32,), jnp.int32),
            pltpu.VMEM((_T,), jnp.int32),
            pltpu.VMEM((_T,), jnp.float32),
            pltpu.VMEM((_NBUF, _C, _D), jnp.float32),
            pltpu.SemaphoreType.DMA,
            pltpu.SemaphoreType.DMA,
            pltpu.SemaphoreType.DMA,
            pltpu.SemaphoreType.DMA,
        ],
    )(xf, shf)
    return out.reshape(_B, _T, _D)
